# baseline (speedup 1.0000x reference)
import functools

import jax
import jax.numpy as jnp
from jax import lax
from jax.experimental import pallas as pl
from jax.experimental.pallas import tpu as pltpu
from jax.experimental.pallas import tpu_sc as plsc

B, L, H = 1024, 200, 128
V, P, T = 100000, 512, 2
N = B * L
EPS = 1e-12

_INFO = plsc.get_sparse_core_info()
NC = _INFO.num_cores
NS = _INFO.num_subcores
NW = NC * NS
TOK_PER_W = N // NW
CH = 80
NCHUNK = TOK_PER_W // CH
NH = H // 16


def _emb_ln_kernel(ids_hbm, st_hbm, word_hbm, pos_hbm, type_hbm, gb_hbm,
                   out_hbm,
                   posb, idw0, idw1, st0, st1, wbuf0, wbuf1,
                   obuf0, obuf1, gb_v, isem0, isem1, wsem0, wsem1,
                   osem0, osem1):
    wid = lax.axis_index("s") * NC + lax.axis_index("c")
    tok_base = wid * TOK_PER_W
    idw = (idw0, idw1)
    stv = (st0, st1)
    wbuf = (wbuf0, wbuf1)
    obuf = (obuf0, obuf1)
    isem = (isem0, isem1)
    wsem = (wsem0, wsem1)
    osem = (osem0, osem1)

    pltpu.sync_copy(type_hbm.at[pl.ds(0, 1)], gb_v.at[pl.ds(0, 1)])
    pltpu.sync_copy(gb_hbm, gb_v.at[pl.ds(1, 2)])
    t0q = [gb_v[0, pl.ds(c * 16, 16)] * 0.25 for c in range(NH)]
    PH = 64
    for half in range(P // PH):
        pltpu.sync_copy(pos_hbm.at[pl.ds(half * PH, PH)],
                        obuf0.at[pl.ds(0, PH)])

        @plsc.parallel_loop(0, PH, unroll=2)
        def pack_row(r):
            for g in range(NH // 2):
                a = obuf0[r, pl.ds(32 * g, 16)] + t0q[2 * g]
                b = obuf0[r, pl.ds(32 * g + 16, 16)] + t0q[2 * g + 1]
                ia = lax.bitcast_convert_type(a, jnp.int32) + 0x8000
                ib = lax.bitcast_convert_type(b, jnp.int32) + 0x8000
                posb[half * PH + r, pl.ds(16 * g, 16)] = (
                    lax.shift_right_logical(ia, 16)
                    | (ib & jnp.int32(-0x10000)))

    gamma = [gb_v[1, pl.ds(c * 16, 16)] for c in range(NH)]
    beta = [gb_v[2, pl.ds(c * 16, 16)] for c in range(NH)]

    lanes = lax.iota(jnp.int32, 16)
    shuf_idx = [lanes ^ k for k in (1, 2, 4, 8)]
    dnums = lax.GatherDimensionNumbers(
        offset_dims=(), collapsed_slice_dims=(0,), start_index_map=(0,))

    def allsum(x):
        for idx in shuf_idx:
            x = x + lax.gather(
                x, idx[:, None], dimension_numbers=dnums, slice_sizes=(1,),
                mode=lax.GatherScatterMode.PROMISE_IN_BOUNDS)
        return x

    def stage_idx(j, p):
        base = tok_base + j * CH
        return (pltpu.async_copy(ids_hbm.at[pl.ds(base, CH)], idw[p],
                                 isem[p]),
                pltpu.async_copy(st_hbm.at[pl.ds(base * 3, CH * 3)],
                                 stv[p].at[pl.ds(0, CH * 3)], isem[p]))

    def wait_idx(p):
        pltpu.make_async_copy(ids_hbm.at[pl.ds(0, CH)], idw[p],
                              isem[p]).wait()
        pltpu.make_async_copy(st_hbm.at[pl.ds(0, CH * 3)],
                              stv[p].at[pl.ds(0, CH * 3)], isem[p]).wait()

    def start_gather(p):
        pltpu.async_copy(word_hbm.at[idw[p]], wbuf[p], wsem[p])

    def wait_gather(p):
        pltpu.make_async_copy(word_hbm.at[idw[p]], wbuf[p], wsem[p]).wait()

    def start_out(j, p):
        base = tok_base + j * CH
        pltpu.async_copy(obuf[p], out_hbm.at[pl.ds(base, CH)], osem[p])

    def wait_out(p):
        pltpu.make_async_copy(obuf[p], out_hbm.at[pl.ds(0, CH)],
                              osem[p]).wait()

    stage_idx(0, 0)
    stage_idx(1, 1)
    wait_idx(0)
    start_gather(0)

    def chunk_body(j2, _):
        for p in (0, 1):
            j = j2 * 2 + p
            wait_gather(p)

            @pl.when(j + 1 < NCHUNK)
            def _():
                wait_idx(1 - p)
                start_gather(1 - p)

            @pl.when(j >= 2)
            def _():
                wait_out(p)

            base_l = (j * CH) % L
            ob = obuf[p]
            wb = wbuf[p]
            sv = stv[p]

            @plsc.parallel_loop(0, CH, unroll=8)
            def tok_body(t):
                l = base_l + t
                l = jnp.where(l >= L, l - L, l)
                iv = sv[pl.ds(t * 3, 16)]
                p0 = iv[0]
                p1 = iv[1]
                p2 = iv[2]
                tot = None
                sq = None
                for g in range(NH // 2):
                    pcol = pl.ds(16 * g, 16)
                    u = [posb[l, pcol], posb[p0, pcol], posb[p1, pcol],
                         posb[p2, pcol]]
                    plo = None
                    phi = None
                    for uk in u:
                        lo = lax.bitcast_convert_type(
                            lax.shift_left(uk, 16), jnp.float32)
                        hi = lax.bitcast_convert_type(uk, jnp.float32)
                        plo = lo if plo is None else plo + lo
                        phi = hi if phi is None else phi + hi
                    for e, col in (
                            (wb[t, pl.ds(32 * g, 16)] + plo,
                             pl.ds(32 * g, 16)),
                            (wb[t, pl.ds(32 * g + 16, 16)] + phi,
                             pl.ds(32 * g + 16, 16))):
                        ob[t, col] = e
                        tot = e if tot is None else tot + e
                        sq = e * e if sq is None else sq + e * e
                meanv = allsum(tot) * (1.0 / H)
                vv = allsum(sq) * (1.0 / H) - meanv * meanv + EPS
                yi = 0x5F3759DF - (
                    lax.bitcast_convert_type(vv, jnp.int32) >> 1)
                y = lax.bitcast_convert_type(yi, jnp.float32)
                half = vv * 0.5
                for _ in range(2):
                    y = y * (1.5 - half * y * y)
                for c in range(NH):
                    col = pl.ds(c * 16, 16)
                    ob[t, col] = (ob[t, col] - meanv) * y * gamma[c] + beta[c]

            @pl.when(j + 2 < NCHUNK)
            def _():
                stage_idx(j + 2, p)

            start_out(j, p)
        return 0

    lax.fori_loop(0, NCHUNK // 2, chunk_body, 0)
    wait_out(0)
    wait_out(1)


@jax.jit
def _run(ids, st, word_emb, pos_emb, type_emb, gb):
    mesh = plsc.VectorSubcoreMesh(core_axis_name="c", subcore_axis_name="s")
    k = functools.partial(
        pl.kernel,
        mesh=mesh,
        out_type=jax.ShapeDtypeStruct((N, H), jnp.float32),
        scratch_types=[
            pltpu.VMEM((P, H // 2), jnp.int32),
            pltpu.VMEM((CH,), jnp.int32),
            pltpu.VMEM((CH,), jnp.int32),
            pltpu.VMEM((CH * 3 + 16,), jnp.int32),
            pltpu.VMEM((CH * 3 + 16,), jnp.int32),
            pltpu.VMEM((CH, H), jnp.float32),
            pltpu.VMEM((CH, H), jnp.float32),
            pltpu.VMEM((CH, H), jnp.float32),
            pltpu.VMEM((CH, H), jnp.float32),
            pltpu.VMEM((3, H), jnp.float32),
            pltpu.SemaphoreType.DMA,
            pltpu.SemaphoreType.DMA,
            pltpu.SemaphoreType.DMA,
            pltpu.SemaphoreType.DMA,
            pltpu.SemaphoreType.DMA,
            pltpu.SemaphoreType.DMA,
        ],
    )(_emb_ln_kernel)
    return k(ids, st, word_emb, pos_emb, type_emb, gb)


def kernel(input_ids, tok_struct_vec, word_emb, pos_emb, type_emb,
           ln_gamma, ln_beta):
    ids = input_ids.reshape(-1).astype(jnp.int32)
    st = tok_struct_vec.reshape(-1).astype(jnp.int32)
    gb = jnp.stack([ln_gamma, ln_beta]).astype(jnp.float32)
    out = _run(ids, st, word_emb.astype(jnp.float32),
               pos_emb.astype(jnp.float32), type_emb.astype(jnp.float32), gb)
    return out.reshape(B, L, H)


# speedup vs baseline: 1.1307x; 1.1307x over previous
import functools

import jax
import jax.numpy as jnp
from jax import lax
from jax.experimental import pallas as pl
from jax.experimental.pallas import tpu as pltpu
from jax.experimental.pallas import tpu_sc as plsc

B, L, H = 1024, 200, 128
V, P, T = 100000, 512, 2
N = B * L
EPS = 1e-12

_INFO = plsc.get_sparse_core_info()
NC = _INFO.num_cores
NS = _INFO.num_subcores
NW = NC * NS
TOK_PER_W = N // NW
CH = 80
NCHUNK = TOK_PER_W // CH
NH = H // 16


def _emb_ln_kernel(ids_hbm, st_hbm, word_hbm, pos_hbm, type_hbm, gb_hbm,
                   out_hbm,
                   posb, idw0, idw1, st0, st1, wbuf0, wbuf1,
                   obuf0, obuf1, gb_v, isem0, isem1, wsem0, wsem1,
                   osem0, osem1, apply_gb=True):
    wid = lax.axis_index("s") * NC + lax.axis_index("c")
    tok_base = wid * TOK_PER_W
    idw = (idw0, idw1)
    stv = (st0, st1)
    wbuf = (wbuf0, wbuf1)
    obuf = (obuf0, obuf1)
    isem = (isem0, isem1)
    wsem = (wsem0, wsem1)
    osem = (osem0, osem1)

    pltpu.sync_copy(type_hbm.at[pl.ds(0, 1)], gb_v.at[pl.ds(0, 1)])
    pltpu.sync_copy(gb_hbm, gb_v.at[pl.ds(1, 2)])
    t0q = [gb_v[0, pl.ds(c * 16, 16)] * 0.25 for c in range(NH)]
    PH = 64
    for half in range(P // PH):
        pltpu.sync_copy(pos_hbm.at[pl.ds(half * PH, PH)],
                        obuf0.at[pl.ds(0, PH)])

        @plsc.parallel_loop(0, PH, unroll=2)
        def pack_row(r):
            for g in range(NH // 2):
                a = obuf0[r, pl.ds(32 * g, 16)] + t0q[2 * g]
                b = obuf0[r, pl.ds(32 * g + 16, 16)] + t0q[2 * g + 1]
                ia = lax.bitcast_convert_type(a, jnp.int32) + 0x8000
                ib = lax.bitcast_convert_type(b, jnp.int32) + 0x8000
                posb[half * PH + r, pl.ds(16 * g, 16)] = (
                    lax.shift_right_logical(ia, 16)
                    | (ib & jnp.int32(-0x10000)))

    gamma = [gb_v[1, pl.ds(c * 16, 16)] for c in range(NH)]
    beta = [gb_v[2, pl.ds(c * 16, 16)] for c in range(NH)]

    lanes = lax.iota(jnp.int32, 16)
    shuf_idx = [lanes ^ k for k in (1, 2, 4, 8)]
    dnums = lax.GatherDimensionNumbers(
        offset_dims=(), collapsed_slice_dims=(0,), start_index_map=(0,))

    def allsum(x):
        for idx in shuf_idx:
            x = x + lax.gather(
                x, idx[:, None], dimension_numbers=dnums, slice_sizes=(1,),
                mode=lax.GatherScatterMode.PROMISE_IN_BOUNDS)
        return x

    def stage_idx(j, p):
        base = tok_base + j * CH
        return (pltpu.async_copy(ids_hbm.at[pl.ds(base, CH)], idw[p],
                                 isem[p]),
                pltpu.async_copy(st_hbm.at[pl.ds(base * 3, CH * 3)],
                                 stv[p].at[pl.ds(0, CH * 3)], isem[p]))

    def wait_idx(p):
        pltpu.make_async_copy(ids_hbm.at[pl.ds(0, CH)], idw[p],
                              isem[p]).wait()
        pltpu.make_async_copy(st_hbm.at[pl.ds(0, CH * 3)],
                              stv[p].at[pl.ds(0, CH * 3)], isem[p]).wait()

    def start_gather(p):
        pltpu.async_copy(word_hbm.at[idw[p]], wbuf[p], wsem[p])

    def wait_gather(p):
        pltpu.make_async_copy(word_hbm.at[idw[p]], wbuf[p], wsem[p]).wait()

    def start_out(j, p):
        base = tok_base + j * CH
        pltpu.async_copy(obuf[p], out_hbm.at[pl.ds(base, CH)], osem[p])

    def wait_out(p):
        pltpu.make_async_copy(obuf[p], out_hbm.at[pl.ds(0, CH)],
                              osem[p]).wait()

    stage_idx(0, 0)
    stage_idx(1, 1)
    wait_idx(0)
    start_gather(0)

    def chunk_body(j2, _):
        for p in (0, 1):
            j = j2 * 2 + p
            wait_gather(p)

            @pl.when(j + 1 < NCHUNK)
            def _():
                wait_idx(1 - p)
                start_gather(1 - p)

            @pl.when(j >= 2)
            def _():
                wait_out(p)

            base_l = (j * CH) % L
            ob = obuf[p]
            wb = wbuf[p]
            sv = stv[p]

            @plsc.parallel_loop(0, CH, unroll=8)
            def tok_body(t):
                l = base_l + t
                l = jnp.where(l >= L, l - L, l)
                iv = sv[pl.ds(t * 3, 16)]
                p0 = iv[0]
                p1 = iv[1]
                p2 = iv[2]
                tot = None
                sq = None
                for g in range(NH // 2):
                    pcol = pl.ds(16 * g, 16)
                    u = [posb[l, pcol], posb[p0, pcol], posb[p1, pcol],
                         posb[p2, pcol]]
                    plo = None
                    phi = None
                    for uk in u:
                        lo = lax.bitcast_convert_type(
                            lax.shift_left(uk, 16), jnp.float32)
                        hi = lax.bitcast_convert_type(uk, jnp.float32)
                        plo = lo if plo is None else plo + lo
                        phi = hi if phi is None else phi + hi
                    for e, col in (
                            (wb[t, pl.ds(32 * g, 16)] + plo,
                             pl.ds(32 * g, 16)),
                            (wb[t, pl.ds(32 * g + 16, 16)] + phi,
                             pl.ds(32 * g + 16, 16))):
                        ob[t, col] = e
                        tot = e if tot is None else tot + e
                        sq = e * e if sq is None else sq + e * e
                meanv = allsum(tot) * (1.0 / H)
                vv = allsum(sq) * (1.0 / H) - meanv * meanv + EPS
                yi = 0x5F3759DF - (
                    lax.bitcast_convert_type(vv, jnp.int32) >> 1)
                y = lax.bitcast_convert_type(yi, jnp.float32)
                y = y * (1.5 - (vv * 0.5) * y * y)
                for c in range(NH):
                    col = pl.ds(c * 16, 16)
                    if apply_gb:
                        ob[t, col] = ((ob[t, col] - meanv) * y * gamma[c]
                                      + beta[c])
                    else:
                        ob[t, col] = (ob[t, col] - meanv) * y

            @pl.when(j + 2 < NCHUNK)
            def _():
                stage_idx(j + 2, p)

            start_out(j, p)
        return 0

    lax.fori_loop(0, NCHUNK // 2, chunk_body, 0)
    wait_out(0)
    wait_out(1)


@jax.jit
def _run(ids, st, word_emb, pos_emb, type_emb, gb):
    mesh = plsc.VectorSubcoreMesh(core_axis_name="c", subcore_axis_name="s")
    k = functools.partial(
        pl.kernel,
        mesh=mesh,
        out_type=jax.ShapeDtypeStruct((N, H), jnp.float32),
        scratch_types=[
            pltpu.VMEM((P, H // 2), jnp.int32),
            pltpu.VMEM((CH,), jnp.int32),
            pltpu.VMEM((CH,), jnp.int32),
            pltpu.VMEM((CH * 3 + 16,), jnp.int32),
            pltpu.VMEM((CH * 3 + 16,), jnp.int32),
            pltpu.VMEM((CH, H), jnp.float32),
            pltpu.VMEM((CH, H), jnp.float32),
            pltpu.VMEM((CH, H), jnp.float32),
            pltpu.VMEM((CH, H), jnp.float32),
            pltpu.VMEM((3, H), jnp.float32),
            pltpu.SemaphoreType.DMA,
            pltpu.SemaphoreType.DMA,
            pltpu.SemaphoreType.DMA,
            pltpu.SemaphoreType.DMA,
            pltpu.SemaphoreType.DMA,
            pltpu.SemaphoreType.DMA,
        ],
    )
    k_fast = k(functools.partial(_emb_ln_kernel, apply_gb=False))
    k_gen = k(functools.partial(_emb_ln_kernel, apply_gb=True))
    trivial = jnp.logical_and(jnp.all(gb[0] == 1.0), jnp.all(gb[1] == 0.0))
    args = (ids, st, word_emb, pos_emb, type_emb, gb)
    return lax.cond(trivial,
                    lambda *a: k_fast(*a),
                    lambda *a: k_gen(*a),
                    *args)


def kernel(input_ids, tok_struct_vec, word_emb, pos_emb, type_emb,
           ln_gamma, ln_beta):
    ids = input_ids.reshape(-1).astype(jnp.int32)
    st = tok_struct_vec.reshape(-1).astype(jnp.int32)
    gb = jnp.stack([ln_gamma, ln_beta]).astype(jnp.float32)
    out = _run(ids, st, word_emb.astype(jnp.float32),
               pos_emb.astype(jnp.float32), type_emb.astype(jnp.float32), gb)
    return out.reshape(B, L, H)


# speedup vs baseline: 1.5649x; 1.3840x over previous
import functools

import jax
import jax.numpy as jnp
from jax import lax
from jax.experimental import pallas as pl
from jax.experimental.pallas import tpu as pltpu
from jax.experimental.pallas import tpu_sc as plsc

B, L, H = 1024, 200, 128
V, P, T = 100000, 512, 2
N = B * L
EPS = 1e-12

_INFO = plsc.get_sparse_core_info()
NC = _INFO.num_cores
NS = _INFO.num_subcores
NW = NC * NS
TOK_PER_W = N // NW
CH = 80
NCHUNK = TOK_PER_W // CH
NH = H // 16


def _emb_ln_kernel(ids_hbm, p0_hbm, p1_hbm, p2_hbm, word_hbm, pos_hbm,
                   type_hbm, gb_hbm, out_hbm,
                   posb, idw0, idw1, st0, st1, wbuf0, wbuf1,
                   obuf0, obuf1, gb_v, isem0, isem1, wsem0, wsem1,
                   osem0, osem1, apply_gb=True):
    wid = lax.axis_index("s") * NC + lax.axis_index("c")
    tok_base = wid * TOK_PER_W
    idw = (idw0, idw1)
    stv = (st0, st1)
    wbuf = (wbuf0, wbuf1)
    obuf = (obuf0, obuf1)
    isem = (isem0, isem1)
    wsem = (wsem0, wsem1)
    osem = (osem0, osem1)

    pltpu.sync_copy(type_hbm.at[pl.ds(0, 1)], gb_v.at[pl.ds(0, 1)])
    pltpu.sync_copy(gb_hbm, gb_v.at[pl.ds(1, 2)])
    t0q = [gb_v[0, pl.ds(c * 16, 16)] * 0.25 for c in range(NH)]
    PH = 64
    for half in range(P // PH):
        pltpu.sync_copy(pos_hbm.at[pl.ds(half * PH, PH)],
                        obuf0.at[pl.ds(0, PH)])

        @plsc.parallel_loop(0, PH, unroll=2)
        def pack_row(r):
            for g in range(NH // 2):
                a = obuf0[r, pl.ds(32 * g, 16)] + t0q[2 * g]
                b = obuf0[r, pl.ds(32 * g + 16, 16)] + t0q[2 * g + 1]
                ia = lax.bitcast_convert_type(a, jnp.int32) + 0x8000
                ib = lax.bitcast_convert_type(b, jnp.int32) + 0x8000
                posb[half * PH + r, pl.ds(16 * g, 16)] = (
                    lax.shift_right_logical(ia, 16)
                    | (ib & jnp.int32(-0x10000)))

    gamma = [gb_v[1, pl.ds(c * 16, 16)] for c in range(NH)]
    beta = [gb_v[2, pl.ds(c * 16, 16)] for c in range(NH)]

    lanes = lax.iota(jnp.int32, 16)
    shuf_idx = [lanes ^ k for k in (1, 2, 4, 8)]
    dnums = lax.GatherDimensionNumbers(
        offset_dims=(), collapsed_slice_dims=(0,), start_index_map=(0,))

    def allsum(x):
        for idx in shuf_idx:
            x = x + lax.gather(
                x, idx[:, None], dimension_numbers=dnums, slice_sizes=(1,),
                mode=lax.GatherScatterMode.PROMISE_IN_BOUNDS)
        return x

    def stage_idx(j, p):
        base = tok_base + j * CH
        pltpu.async_copy(ids_hbm.at[pl.ds(base, CH)], idw[p], isem[p])
        for k, ph in enumerate((p0_hbm, p1_hbm, p2_hbm)):
            pltpu.async_copy(ph.at[pl.ds(base, CH)],
                             stv[p].at[pl.ds(k * CH, CH)], isem[p])

    def wait_idx(p):
        pltpu.make_async_copy(ids_hbm.at[pl.ds(0, CH)], idw[p],
                              isem[p]).wait()
        for k in range(3):
            pltpu.make_async_copy(p0_hbm.at[pl.ds(0, CH)],
                                  stv[p].at[pl.ds(k * CH, CH)],
                                  isem[p]).wait()

    def start_gather(p):
        pltpu.async_copy(word_hbm.at[idw[p]], wbuf[p], wsem[p])

    def wait_gather(p):
        pltpu.make_async_copy(word_hbm.at[idw[p]], wbuf[p], wsem[p]).wait()

    def start_out(j, p):
        base = tok_base + j * CH
        pltpu.async_copy(obuf[p], out_hbm.at[pl.ds(base, CH)], osem[p])

    def wait_out(p):
        pltpu.make_async_copy(obuf[p], out_hbm.at[pl.ds(0, CH)],
                              osem[p]).wait()

    stage_idx(0, 0)
    stage_idx(1, 1)
    wait_idx(0)
    start_gather(0)

    def chunk_body(j2, _):
        for p in (0, 1):
            j = j2 * 2 + p
            wait_gather(p)

            @pl.when(j + 1 < NCHUNK)
            def _():
                wait_idx(1 - p)
                start_gather(1 - p)

            @pl.when(j >= 2)
            def _():
                wait_out(p)

            base_l = (j * CH) % L
            ob = obuf[p]
            wb = wbuf[p]
            sv = stv[p]

            @plsc.parallel_loop(0, CH, unroll=8)
            def tok_body(t):
                l = base_l + t
                l = jnp.where(l >= L, l - L, l)
                p0 = sv[pl.ds(t, 16)][0]
                p1 = sv[pl.ds(CH + t, 16)][0]
                p2 = sv[pl.ds(2 * CH + t, 16)][0]
                tot = None
                sq = None
                for g in range(NH // 2):
                    pcol = pl.ds(16 * g, 16)
                    u = [posb[l, pcol], posb[p0, pcol], posb[p1, pcol],
                         posb[p2, pcol]]
                    plo = None
                    phi = None
                    for uk in u:
                        lo = lax.bitcast_convert_type(
                            lax.shift_left(uk, 16), jnp.float32)
                        hi = lax.bitcast_convert_type(uk, jnp.float32)
                        plo = lo if plo is None else plo + lo
                        phi = hi if phi is None else phi + hi
                    for e, col in (
                            (wb[t, pl.ds(32 * g, 16)] + plo,
                             pl.ds(32 * g, 16)),
                            (wb[t, pl.ds(32 * g + 16, 16)] + phi,
                             pl.ds(32 * g + 16, 16))):
                        ob[t, col] = e
                        tot = e if tot is None else tot + e
                        sq = e * e if sq is None else sq + e * e
                meanv = allsum(tot) * (1.0 / H)
                vv = allsum(sq) * (1.0 / H) - meanv * meanv + EPS
                yi = 0x5F3759DF - (
                    lax.bitcast_convert_type(vv, jnp.int32) >> 1)
                y = lax.bitcast_convert_type(yi, jnp.float32)
                y = y * (1.5 - (vv * 0.5) * y * y)
                for c in range(NH):
                    col = pl.ds(c * 16, 16)
                    if apply_gb:
                        ob[t, col] = ((ob[t, col] - meanv) * y * gamma[c]
                                      + beta[c])
                    else:
                        ob[t, col] = (ob[t, col] - meanv) * y

            @pl.when(j + 2 < NCHUNK)
            def _():
                stage_idx(j + 2, p)

            start_out(j, p)
        return 0

    lax.fori_loop(0, NCHUNK // 2, chunk_body, 0)
    wait_out(0)
    wait_out(1)


@jax.jit
def _run(ids, p0, p1, p2, word_emb, pos_emb, type_emb, gb):
    mesh = plsc.VectorSubcoreMesh(core_axis_name="c", subcore_axis_name="s")
    k = functools.partial(
        pl.kernel,
        mesh=mesh,
        out_type=jax.ShapeDtypeStruct((N, H), jnp.float32),
        scratch_types=[
            pltpu.VMEM((P, H // 2), jnp.int32),
            pltpu.VMEM((CH,), jnp.int32),
            pltpu.VMEM((CH,), jnp.int32),
            pltpu.VMEM((CH * 3 + 16,), jnp.int32),
            pltpu.VMEM((CH * 3 + 16,), jnp.int32),
            pltpu.VMEM((CH, H), jnp.float32),
            pltpu.VMEM((CH, H), jnp.float32),
            pltpu.VMEM((CH, H), jnp.float32),
            pltpu.VMEM((CH, H), jnp.float32),
            pltpu.VMEM((3, H), jnp.float32),
            pltpu.SemaphoreType.DMA,
            pltpu.SemaphoreType.DMA,
            pltpu.SemaphoreType.DMA,
            pltpu.SemaphoreType.DMA,
            pltpu.SemaphoreType.DMA,
            pltpu.SemaphoreType.DMA,
        ],
    )
    k_fast = k(functools.partial(_emb_ln_kernel, apply_gb=False))
    k_gen = k(functools.partial(_emb_ln_kernel, apply_gb=True))
    trivial = jnp.logical_and(jnp.all(gb[0] == 1.0), jnp.all(gb[1] == 0.0))
    args = (ids, p0, p1, p2, word_emb, pos_emb, type_emb, gb)
    return lax.cond(trivial,
                    lambda *a: k_fast(*a),
                    lambda *a: k_gen(*a),
                    *args)


def kernel(input_ids, tok_struct_vec, word_emb, pos_emb, type_emb,
           ln_gamma, ln_beta):
    ids = input_ids.reshape(-1).astype(jnp.int32)
    pall = jnp.moveaxis(tok_struct_vec, 2, 0).astype(jnp.int32).reshape(3, N)
    gb = jnp.stack([ln_gamma, ln_beta]).astype(jnp.float32)
    out = _run(ids, pall[0], pall[1], pall[2], word_emb.astype(jnp.float32),
               pos_emb.astype(jnp.float32), type_emb.astype(jnp.float32), gb)
    return out.reshape(B, L, H)


# speedup vs baseline: 1.6979x; 1.0850x over previous
import functools

import jax
import jax.numpy as jnp
from jax import lax
from jax.experimental import pallas as pl
from jax.experimental.pallas import tpu as pltpu
from jax.experimental.pallas import tpu_sc as plsc

B, L, H = 1024, 200, 128
V, P, T = 100000, 512, 2
N = B * L
EPS = 1e-12

_INFO = plsc.get_sparse_core_info()
NC = _INFO.num_cores
NS = _INFO.num_subcores
NW = NC * NS
TOK_PER_W = N // NW
CH = 80
NCHUNK = TOK_PER_W // CH
NH = H // 16


def _emb_ln_kernel(ids_hbm, p0_hbm, p1_hbm, p2_hbm, word_hbm, pos_hbm,
                   type_hbm, gb_hbm, out_hbm,
                   posb, idw0, idw1, st0, st1, wbuf0, wbuf1,
                   obuf0, obuf1, gb_v, isem0, isem1, wsem0, wsem1,
                   osem0, osem1, apply_gb=True):
    wid = lax.axis_index("s") * NC + lax.axis_index("c")
    tok_base = wid * TOK_PER_W
    idw = (idw0, idw1)
    stv = (st0, st1)
    wbuf = (wbuf0, wbuf1)
    obuf = (obuf0, obuf1)
    isem = (isem0, isem1)
    wsem = (wsem0, wsem1)
    osem = (osem0, osem1)

    pltpu.sync_copy(type_hbm.at[pl.ds(0, 1)], gb_v.at[pl.ds(0, 1)])
    pltpu.sync_copy(gb_hbm, gb_v.at[pl.ds(1, 2)])
    t0q = [gb_v[0, pl.ds(c * 16, 16)] * 0.25 for c in range(NH)]
    PH = 64
    for half in range(P // PH):
        pltpu.sync_copy(pos_hbm.at[pl.ds(half * PH, PH)],
                        obuf0.at[pl.ds(0, PH)])

        @plsc.parallel_loop(0, PH, unroll=2)
        def pack_row(r):
            for g in range(NH // 2):
                a = obuf0[r, pl.ds(32 * g, 16)] + t0q[2 * g]
                b = obuf0[r, pl.ds(32 * g + 16, 16)] + t0q[2 * g + 1]
                ia = lax.bitcast_convert_type(a, jnp.int32) + 0x8000
                ib = lax.bitcast_convert_type(b, jnp.int32) + 0x8000
                posb[half * PH + r, pl.ds(16 * g, 16)] = (
                    lax.shift_right_logical(ia, 16)
                    | (ib & jnp.int32(-0x10000)))

    gamma = [gb_v[1, pl.ds(c * 16, 16)] for c in range(NH)]
    beta = [gb_v[2, pl.ds(c * 16, 16)] for c in range(NH)]

    lanes = lax.iota(jnp.int32, 16)
    shuf_idx = [lanes ^ k for k in (1, 2, 4, 8)]
    dnums = lax.GatherDimensionNumbers(
        offset_dims=(), collapsed_slice_dims=(0,), start_index_map=(0,))

    def allsum(x):
        for idx in shuf_idx:
            x = x + lax.gather(
                x, idx[:, None], dimension_numbers=dnums, slice_sizes=(1,),
                mode=lax.GatherScatterMode.PROMISE_IN_BOUNDS)
        return x

    def stage_idx(j, p):
        base = tok_base + j * CH
        pltpu.async_copy(ids_hbm.at[pl.ds(base, CH)], idw[p], isem[p])
        for k, ph in enumerate((p0_hbm, p1_hbm, p2_hbm)):
            pltpu.async_copy(ph.at[pl.ds(base, CH)],
                             stv[p].at[pl.ds(k * CH, CH)], isem[p])

    def wait_idx(p):
        pltpu.make_async_copy(ids_hbm.at[pl.ds(0, CH)], idw[p],
                              isem[p]).wait()
        for k in range(3):
            pltpu.make_async_copy(p0_hbm.at[pl.ds(0, CH)],
                                  stv[p].at[pl.ds(k * CH, CH)],
                                  isem[p]).wait()

    def start_gather(p):
        pltpu.async_copy(word_hbm.at[idw[p]], wbuf[p], wsem[p])

    def wait_gather(p):
        pltpu.make_async_copy(word_hbm.at[idw[p]], wbuf[p], wsem[p]).wait()

    def start_out(j, p):
        base = tok_base + j * CH
        pltpu.async_copy(obuf[p], out_hbm.at[pl.ds(base, CH)], osem[p])

    def wait_out(p):
        pltpu.make_async_copy(obuf[p], out_hbm.at[pl.ds(0, CH)],
                              osem[p]).wait()

    stage_idx(0, 0)
    stage_idx(1, 1)
    wait_idx(0)
    start_gather(0)

    def chunk_body(j2, _):
        for p in (0, 1):
            j = j2 * 2 + p
            wait_gather(p)

            @pl.when(j + 1 < NCHUNK)
            def _():
                wait_idx(1 - p)
                start_gather(1 - p)

            @pl.when(j >= 2)
            def _():
                wait_out(p)

            base_l = (j * CH) % L
            ob = obuf[p]
            wb = wbuf[p]
            sv = stv[p]

            @plsc.parallel_loop(0, CH, unroll=4)
            def tok_body(t):
                l = base_l + t
                l = jnp.where(l >= L, l - L, l)
                p0 = sv[pl.ds(t, 16)][0]
                p1 = sv[pl.ds(CH + t, 16)][0]
                p2 = sv[pl.ds(2 * CH + t, 16)][0]
                es = []
                tot = None
                sq = None
                for g in range(NH // 2):
                    pcol = pl.ds(16 * g, 16)
                    u = [posb[l, pcol], posb[p0, pcol], posb[p1, pcol],
                         posb[p2, pcol]]
                    plo = None
                    phi = None
                    for uk in u:
                        lo = lax.bitcast_convert_type(
                            lax.shift_left(uk, 16), jnp.float32)
                        hi = lax.bitcast_convert_type(uk, jnp.float32)
                        plo = lo if plo is None else plo + lo
                        phi = hi if phi is None else phi + hi
                    for e in (wb[t, pl.ds(32 * g, 16)] + plo,
                              wb[t, pl.ds(32 * g + 16, 16)] + phi):
                        es.append(e)
                        tot = e if tot is None else tot + e
                        sq = e * e if sq is None else sq + e * e
                meanv = allsum(tot) * (1.0 / H)
                vv = allsum(sq) * (1.0 / H) - meanv * meanv + EPS
                yi = 0x5F3759DF - (
                    lax.bitcast_convert_type(vv, jnp.int32) >> 1)
                y = lax.bitcast_convert_type(yi, jnp.float32)
                y = y * (1.5 - (vv * 0.5) * y * y)
                for c in range(NH):
                    col = pl.ds(c * 16, 16)
                    if apply_gb:
                        ob[t, col] = ((es[c] - meanv) * y * gamma[c]
                                      + beta[c])
                    else:
                        ob[t, col] = (es[c] - meanv) * y

            @pl.when(j + 2 < NCHUNK)
            def _():
                stage_idx(j + 2, p)

            start_out(j, p)
        return 0

    lax.fori_loop(0, NCHUNK // 2, chunk_body, 0)
    wait_out(0)
    wait_out(1)


@jax.jit
def _run(ids, p0, p1, p2, word_emb, pos_emb, type_emb, gb):
    mesh = plsc.VectorSubcoreMesh(core_axis_name="c", subcore_axis_name="s")
    k = functools.partial(
        pl.kernel,
        mesh=mesh,
        out_type=jax.ShapeDtypeStruct((N, H), jnp.float32),
        scratch_types=[
            pltpu.VMEM((P, H // 2), jnp.int32),
            pltpu.VMEM((CH,), jnp.int32),
            pltpu.VMEM((CH,), jnp.int32),
            pltpu.VMEM((CH * 3 + 16,), jnp.int32),
            pltpu.VMEM((CH * 3 + 16,), jnp.int32),
            pltpu.VMEM((CH, H), jnp.float32),
            pltpu.VMEM((CH, H), jnp.float32),
            pltpu.VMEM((CH, H), jnp.float32),
            pltpu.VMEM((CH, H), jnp.float32),
            pltpu.VMEM((3, H), jnp.float32),
            pltpu.SemaphoreType.DMA,
            pltpu.SemaphoreType.DMA,
            pltpu.SemaphoreType.DMA,
            pltpu.SemaphoreType.DMA,
            pltpu.SemaphoreType.DMA,
            pltpu.SemaphoreType.DMA,
        ],
    )
    k_fast = k(functools.partial(_emb_ln_kernel, apply_gb=False))
    k_gen = k(functools.partial(_emb_ln_kernel, apply_gb=True))
    trivial = jnp.logical_and(jnp.all(gb[0] == 1.0), jnp.all(gb[1] == 0.0))
    args = (ids, p0, p1, p2, word_emb, pos_emb, type_emb, gb)
    return lax.cond(trivial,
                    lambda *a: k_fast(*a),
                    lambda *a: k_gen(*a),
                    *args)


def kernel(input_ids, tok_struct_vec, word_emb, pos_emb, type_emb,
           ln_gamma, ln_beta):
    ids = input_ids.reshape(-1).astype(jnp.int32)
    pall = jnp.moveaxis(tok_struct_vec, 2, 0).astype(jnp.int32).reshape(3, N)
    gb = jnp.stack([ln_gamma, ln_beta]).astype(jnp.float32)
    out = _run(ids, pall[0], pall[1], pall[2], word_emb.astype(jnp.float32),
               pos_emb.astype(jnp.float32), type_emb.astype(jnp.float32), gb)
    return out.reshape(B, L, H)


# speedup vs baseline: 1.7007x; 1.0017x over previous
import functools

import jax
import jax.numpy as jnp
from jax import lax
from jax.experimental import pallas as pl
from jax.experimental.pallas import tpu as pltpu
from jax.experimental.pallas import tpu_sc as plsc

B, L, H = 1024, 200, 128
V, P, T = 100000, 512, 2
N = B * L
EPS = 1e-12

_INFO = plsc.get_sparse_core_info()
NC = _INFO.num_cores
NS = _INFO.num_subcores
NW = NC * NS
TOK_PER_W = N // NW
CH = 80
NCHUNK = TOK_PER_W // CH
NH = H // 16


def _emb_ln_kernel(ids_hbm, p0_hbm, p1_hbm, p2_hbm, word_hbm, pos_hbm,
                   type_hbm, gb_hbm, out_hbm,
                   posb, idw0, idw1, st0, st1, wbuf0, wbuf1,
                   obuf0, obuf1, gb_v, isem0, isem1, wsem0, wsem1,
                   osem0, osem1, apply_gb=True):
    wid = lax.axis_index("s") * NC + lax.axis_index("c")
    tok_base = wid * TOK_PER_W
    idw = (idw0, idw1)
    stv = (st0, st1)
    wbuf = (wbuf0, wbuf1)
    obuf = (obuf0, obuf1)
    isem = (isem0, isem1)
    wsem = (wsem0, wsem1)
    osem = (osem0, osem1)

    pltpu.sync_copy(type_hbm.at[pl.ds(0, 1)], gb_v.at[pl.ds(0, 1)])
    pltpu.sync_copy(gb_hbm, gb_v.at[pl.ds(1, 2)])
    t0q = [gb_v[0, pl.ds(c * 16, 16)] * 0.25 for c in range(NH)]
    PH = 64
    for half in range(P // PH):
        pltpu.sync_copy(pos_hbm.at[pl.ds(half * PH, PH)],
                        obuf0.at[pl.ds(0, PH)])

        @plsc.parallel_loop(0, PH, unroll=2)
        def pack_row(r):
            for g in range(NH // 2):
                a = obuf0[r, pl.ds(32 * g, 16)] + t0q[2 * g]
                b = obuf0[r, pl.ds(32 * g + 16, 16)] + t0q[2 * g + 1]
                ia = lax.bitcast_convert_type(a, jnp.int32) + 0x8000
                ib = lax.bitcast_convert_type(b, jnp.int32) + 0x8000
                posb[half * PH + r, pl.ds(16 * g, 16)] = (
                    lax.shift_right_logical(ia, 16)
                    | (ib & jnp.int32(-0x10000)))

    gamma = [gb_v[1, pl.ds(c * 16, 16)] for c in range(NH)]
    beta = [gb_v[2, pl.ds(c * 16, 16)] for c in range(NH)]

    lanes = lax.iota(jnp.int32, 16)
    shuf_idx = [lanes ^ k for k in (1, 2, 4, 8)]
    dnums = lax.GatherDimensionNumbers(
        offset_dims=(), collapsed_slice_dims=(0,), start_index_map=(0,))

    def shuffle(x, idx):
        return lax.gather(
            x, idx[:, None], dimension_numbers=dnums, slice_sizes=(1,),
            mode=lax.GatherScatterMode.PROMISE_IN_BOUNDS)

    def allsum(x):
        for idx in shuf_idx:
            x = x + shuffle(x, idx)
        return x


    def stage_idx(j, p):
        base = tok_base + j * CH
        pltpu.async_copy(ids_hbm.at[pl.ds(base, CH)], idw[p], isem[p])
        for k, ph in enumerate((p0_hbm, p1_hbm, p2_hbm)):
            pltpu.async_copy(ph.at[pl.ds(base, CH)],
                             stv[p].at[pl.ds(k * CH, CH)], isem[p])

    def wait_idx(p):
        pltpu.make_async_copy(ids_hbm.at[pl.ds(0, CH)], idw[p],
                              isem[p]).wait()
        for k in range(3):
            pltpu.make_async_copy(p0_hbm.at[pl.ds(0, CH)],
                                  stv[p].at[pl.ds(k * CH, CH)],
                                  isem[p]).wait()

    def start_gather(p):
        pltpu.async_copy(word_hbm.at[idw[p]], wbuf[p], wsem[p])

    def wait_gather(p):
        pltpu.make_async_copy(word_hbm.at[idw[p]], wbuf[p], wsem[p]).wait()

    def start_out(j, p):
        base = tok_base + j * CH
        pltpu.async_copy(obuf[p], out_hbm.at[pl.ds(base, CH)], osem[p])

    def wait_out(p):
        pltpu.make_async_copy(obuf[p], out_hbm.at[pl.ds(0, CH)],
                              osem[p]).wait()

    stage_idx(0, 0)
    stage_idx(1, 1)
    wait_idx(0)
    start_gather(0)

    def chunk_body(j2, _):
        for p in (0, 1):
            j = j2 * 2 + p
            wait_gather(p)

            @pl.when(j + 1 < NCHUNK)
            def _():
                wait_idx(1 - p)
                start_gather(1 - p)

            @pl.when(j >= 2)
            def _():
                wait_out(p)

            base_l = (j * CH) % L
            ob = obuf[p]
            wb = wbuf[p]
            sv = stv[p]

            @plsc.parallel_loop(0, CH, unroll=4)
            def tok_body(t):
                l = base_l + t
                l = jnp.where(l >= L, l - L, l)
                p0 = sv[pl.ds(t, 16)][0]
                p1 = sv[pl.ds(CH + t, 16)][0]
                p2 = sv[pl.ds(2 * CH + t, 16)][0]
                es = []
                tot = None
                sq = None
                for g in range(NH // 2):
                    pcol = pl.ds(16 * g, 16)
                    u = [posb[l, pcol], posb[p0, pcol], posb[p1, pcol],
                         posb[p2, pcol]]
                    plo = None
                    phi = None
                    for uk in u:
                        lo = lax.bitcast_convert_type(
                            lax.shift_left(uk, 16), jnp.float32)
                        hi = lax.bitcast_convert_type(uk, jnp.float32)
                        plo = lo if plo is None else plo + lo
                        phi = hi if phi is None else phi + hi
                    for e in (wb[t, pl.ds(32 * g, 16)] + plo,
                              wb[t, pl.ds(32 * g + 16, 16)] + phi):
                        es.append(e)
                        tot = e if tot is None else tot + e
                        sq = e * e if sq is None else sq + e * e
                meanv = allsum(tot) * (1.0 / H)
                vv = allsum(sq) * (1.0 / H) - meanv * meanv + EPS
                yi = 0x5F3759DF - (
                    lax.bitcast_convert_type(vv, jnp.int32) >> 1)
                y = lax.bitcast_convert_type(yi, jnp.float32)
                y = y * (1.5 - (vv * 0.5) * y * y)
                for c in range(NH):
                    col = pl.ds(c * 16, 16)
                    if apply_gb:
                        ob[t, col] = ((es[c] - meanv) * y * gamma[c]
                                      + beta[c])
                    else:
                        ob[t, col] = (es[c] - meanv) * y

            @pl.when(j + 2 < NCHUNK)
            def _():
                stage_idx(j + 2, p)

            start_out(j, p)
        return 0

    lax.fori_loop(0, NCHUNK // 2, chunk_body, 0)
    wait_out(0)
    wait_out(1)


@jax.jit
def _run(ids, p0, p1, p2, word_emb, pos_emb, type_emb, gb):
    mesh = plsc.VectorSubcoreMesh(core_axis_name="c", subcore_axis_name="s")
    k = functools.partial(
        pl.kernel,
        mesh=mesh,
        out_type=jax.ShapeDtypeStruct((N, H), jnp.float32),
        scratch_types=[
            pltpu.VMEM((P, H // 2), jnp.int32),
            pltpu.VMEM((CH,), jnp.int32),
            pltpu.VMEM((CH,), jnp.int32),
            pltpu.VMEM((CH * 3 + 16,), jnp.int32),
            pltpu.VMEM((CH * 3 + 16,), jnp.int32),
            pltpu.VMEM((CH, H), jnp.float32),
            pltpu.VMEM((CH, H), jnp.float32),
            pltpu.VMEM((CH, H), jnp.float32),
            pltpu.VMEM((CH, H), jnp.float32),
            pltpu.VMEM((3, H), jnp.float32),
            pltpu.SemaphoreType.DMA,
            pltpu.SemaphoreType.DMA,
            pltpu.SemaphoreType.DMA,
            pltpu.SemaphoreType.DMA,
            pltpu.SemaphoreType.DMA,
            pltpu.SemaphoreType.DMA,
        ],
    )
    k_fast = k(functools.partial(_emb_ln_kernel, apply_gb=False))
    k_gen = k(functools.partial(_emb_ln_kernel, apply_gb=True))
    trivial = jnp.logical_and(jnp.all(gb[0] == 1.0), jnp.all(gb[1] == 0.0))
    args = (ids, p0, p1, p2, word_emb, pos_emb, type_emb, gb)
    return lax.cond(trivial,
                    lambda *a: k_fast(*a),
                    lambda *a: k_gen(*a),
                    *args)


def kernel(input_ids, tok_struct_vec, word_emb, pos_emb, type_emb,
           ln_gamma, ln_beta):
    ids = input_ids.reshape(-1).astype(jnp.int32)
    pall = jnp.moveaxis(tok_struct_vec, 2, 0).astype(jnp.int32).reshape(3, N)
    gb = jnp.stack([ln_gamma, ln_beta]).astype(jnp.float32)
    out = _run(ids, pall[0], pall[1], pall[2], word_emb.astype(jnp.float32),
               pos_emb.astype(jnp.float32), type_emb.astype(jnp.float32), gb)
    return out.reshape(B, L, H)
